# Initial kernel scaffold; baseline (speedup 1.0000x reference)
import jax
import jax.numpy as jnp
from jax.experimental import pallas as pl


def kernel(input, target, distr):
    raise NotImplementedError("write your pallas kernel here")


# speedup vs baseline: 4.8173x; 4.8173x over previous
import numpy as np

import jax
import jax.numpy as jnp
from jax.experimental import pallas as pl
from jax.experimental.pallas import tpu as pltpu

_NUM_WORDS = 1000
_BATCH = 128
_NUM_NEG = 5
_TINY = np.float32(np.finfo(np.float32).tiny)
_SPAN = np.float32(np.float32(1.0) - _TINY)


def _negloss_body(bits_ref, inp_ref, tgt_ref, logp_ref, out_ref):
    N, B, V = _NUM_NEG, _BATCH, _NUM_WORDS

    b32 = bits_ref[...]
    fb = (b32 >> jnp.uint32(9)) | jnp.uint32(0x3F800000)
    f = jax.lax.bitcast_convert_type(fb, jnp.float32) - jnp.float32(1.0)
    u = jnp.maximum(_TINY, f * _SPAN + _TINY)
    g = -jnp.log(-jnp.log(u))

    t = tgt_ref[...]
    col3 = jax.lax.broadcasted_iota(jnp.int32, (N, B, V), 2)
    tmask = col3 == t[None, :, :]
    logp = logp_ref[...]
    score = jnp.where(tmask, jnp.float32(-1e30), g + logp[None, :, :])

    mx = jnp.max(score, axis=2, keepdims=True)
    idx = jnp.min(jnp.where(score == mx, col3, V), axis=2)

    onehot_s = (col3 == idx[:, :, None]).astype(jnp.float32)
    col2 = jax.lax.broadcasted_iota(jnp.int32, (B, V), 1)
    onehot_t = col2 == t
    hist = jnp.sum(onehot_s, axis=(0, 1)) + jnp.sum(
        onehot_t.astype(jnp.float32), axis=0
    )

    w_t = jnp.sum(jnp.where(onehot_t, hist[None, :], 0.0), axis=1, keepdims=True)
    picked = jnp.sum(jnp.where(onehot_t, inp_ref[...], 0.0), axis=1, keepdims=True)
    num = jnp.sum(w_t * picked)
    den = jnp.sum(w_t)
    out_ref[0, 0] = -num / den


def kernel(input, target, distr):
    B, V = input.shape
    N = _NUM_NEG
    bits = jax.random.bits(jax.random.key(123), (N, B, V), dtype=jnp.uint32)
    p = distr / jnp.sum(distr)
    logp = jnp.log(p + 1e-20).reshape(1, V)
    tgt = target.reshape(B, 1)

    out = pl.pallas_call(
        _negloss_body,
        out_shape=jax.ShapeDtypeStruct((1, 1), jnp.float32),
        out_specs=pl.BlockSpec(memory_space=pltpu.SMEM),
    )(bits, input, tgt, logp)
    return out[0, 0]
